# baseline (speedup 1.0000x reference)
import functools

import jax
import jax.numpy as jnp
from jax.experimental import pallas as pl
from jax.experimental.pallas import tpu as pltpu

CUT2 = 100.0
K = 32
BIG = 1.0e30


def _tc_body(posp_blk, posTp, bat_blk, bat_row, idx_out, w_out, d2_ref, dd_ref, *, R, N, CC):
    NC = N // CC
    g = pl.program_id(0)

    xi = posp_blk[:, 0:1]
    yi = posp_blk[:, 1:2]
    zi = posp_blk[:, 2:3]
    sqi = (xi * xi + yi * yi) + zi * zi
    bi = bat_blk[:, 0:1]
    rowg = g * R + jax.lax.broadcasted_iota(jnp.int32, (R, 1), 0)

    def compute_chunk(c, _):
        cols = pl.ds(c * CC, CC)
        p = posTp[:, cols]
        xj = p[0:1, :]
        yj = p[1:2, :]
        zj = p[2:3, :]
        sqj = (xj * xj + yj * yj) + zj * zj
        gram = jnp.dot(posp_blk[...], p, preferred_element_type=jnp.float32)
        d2 = jnp.maximum((sqi + sqj) - 2.0 * gram, 0.0)
        dx = xi - xj
        dy = yi - yj
        dz = zi - zj
        dd = (dx * dx + dy * dy) + dz * dz
        colid = c * CC + jax.lax.broadcasted_iota(jnp.int32, (R, CC), 1)
        same = bi == bat_row[0:1, cols]
        keep = same & (colid != rowg)
        d2_ref[:, cols] = jnp.where(keep, d2, BIG)
        dd_ref[:, cols] = dd
        return 0

    jax.lax.fori_loop(0, NC, compute_chunk, 0)

    slot = jax.lax.broadcasted_iota(jnp.int32, (R, K), 1)

    def select_one(k, state):
        am_prev, idx_acc, w_acc = state

        def scan_chunk(c, carry):
            mv, am, dv = carry
            cols = pl.ds(c * CC, CC)
            dc = d2_ref[:, cols]
            colid = c * CC + jax.lax.broadcasted_iota(jnp.int32, (R, CC), 1)
            dc = jnp.where(colid == am_prev, BIG, dc)
            d2_ref[:, cols] = dc
            cmin = jnp.min(dc, axis=1, keepdims=True)
            hit = dc == cmin
            carg = jnp.min(jnp.where(hit, colid, N), axis=1, keepdims=True)
            cdd = jnp.min(
                jnp.where(colid == carg, dd_ref[:, cols], BIG), axis=1, keepdims=True
            )
            better = cmin < mv
            return (
                jnp.where(better, cmin, mv),
                jnp.where(better, carg, am),
                jnp.where(better, cdd, dv),
            )

        mv0 = jnp.full((R, 1), BIG, jnp.float32)
        am0 = jnp.full((R, 1), -2, jnp.int32)
        mv, am, dv = jax.lax.fori_loop(0, NC, scan_chunk, (mv0, am0, mv0))

        valid = mv <= CUT2
        iv = jnp.where(valid, am, rowg)
        wv = jnp.where(
            valid & (dv > 0.0), jnp.sqrt(jnp.where(dv > 0.0, dv, 1.0)), 0.0
        )
        here = slot == k
        idx_acc = jnp.where(here, iv, idx_acc)
        w_acc = jnp.where(here, wv, w_acc)
        return am, idx_acc, w_acc

    _, idx_acc, w_acc = jax.lax.fori_loop(
        0,
        K,
        select_one,
        (
            jnp.full((R, 1), -2, jnp.int32),
            jnp.zeros((R, K), jnp.int32),
            jnp.zeros((R, K), jnp.float32),
        ),
    )
    idx_out[...] = idx_acc
    w_out[...] = w_acc


def _build(N, R, CC):
    grid = (N // R,)
    return pl.pallas_call(
        functools.partial(_tc_body, R=R, N=N, CC=CC),
        grid=grid,
        in_specs=[
            pl.BlockSpec((R, 8), lambda i: (i, 0)),
            pl.BlockSpec((8, N), lambda i: (0, 0)),
            pl.BlockSpec((R, 1), lambda i: (i, 0)),
            pl.BlockSpec((1, N), lambda i: (0, 0)),
        ],
        out_specs=[
            pl.BlockSpec((R, K), lambda i: (i, 0)),
            pl.BlockSpec((R, K), lambda i: (i, 0)),
        ],
        out_shape=[
            jax.ShapeDtypeStruct((N, K), jnp.int32),
            jax.ShapeDtypeStruct((N, K), jnp.float32),
        ],
        scratch_shapes=[
            pltpu.VMEM((R, N), jnp.float32),
            pltpu.VMEM((R, N), jnp.float32),
        ],
    )


def kernel(pos, batch):
    N = pos.shape[0]
    R = 256 if N % 256 == 0 else 128
    CC = 128
    batch32 = batch.astype(jnp.int32)
    posp = jnp.pad(pos, ((0, 0), (0, 5)))
    posTp = posp.T
    bat_col = batch32.reshape(N, 1)
    bat_row = batch32.reshape(1, N)
    idx, w = _build(N, R, CC)(posp, posTp, bat_col, bat_row)
    centers = jnp.broadcast_to(jnp.arange(N, dtype=jnp.int32)[:, None], (N, K))
    row = idx.reshape(-1)
    col = centers.reshape(-1)
    edge_index = jnp.stack([row, col]).astype(jnp.int64)
    edge_weight = w.reshape(-1)
    return edge_index, edge_weight


# speedup vs baseline: 8.4979x; 8.4979x over previous
import functools

import jax
import jax.numpy as jnp
from jax.experimental import pallas as pl
from jax.experimental.pallas import tpu as pltpu

CUT2 = 100.0
K = 32
BIG = 1.0e30


def _tc_body(clo_ref, chi_ref, posp_blk, posTp, bat_blk, bat_row, idx_out, w_out,
             d2_ref, dd_ref, *, R, N, CC):
    g = pl.program_id(0)
    c_lo = clo_ref[g]
    c_hi = chi_ref[g]

    xi = posp_blk[:, 0:1]
    yi = posp_blk[:, 1:2]
    zi = posp_blk[:, 2:3]
    sqi = (xi * xi + yi * yi) + zi * zi
    bi = bat_blk[:, 0:1]
    rowg = g * R + jax.lax.broadcasted_iota(jnp.int32, (R, 1), 0)

    def compute_chunk(c, _):
        cols = pl.ds(c * CC, CC)
        p = posTp[:, cols]
        xj = p[0:1, :]
        yj = p[1:2, :]
        zj = p[2:3, :]
        sqj = (xj * xj + yj * yj) + zj * zj
        gram = jnp.dot(posp_blk[...], p, preferred_element_type=jnp.float32)
        d2 = jnp.maximum((sqi + sqj) - 2.0 * gram, 0.0)
        dx = xi - xj
        dy = yi - yj
        dz = zi - zj
        dd = (dx * dx + dy * dy) + dz * dz
        colid = c * CC + jax.lax.broadcasted_iota(jnp.int32, (R, CC), 1)
        same = bi == bat_row[0:1, cols]
        keep = same & (colid != rowg)
        d2_ref[:, cols] = jnp.where(keep, d2, BIG)
        dd_ref[:, cols] = dd
        return 0

    jax.lax.fori_loop(c_lo, c_hi, compute_chunk, 0)

    slot = jax.lax.broadcasted_iota(jnp.int32, (R, K), 1)

    def select_one(k, state):
        am_prev, idx_acc, w_acc = state

        def scan_chunk(c, carry):
            mv, am, dv = carry
            cols = pl.ds(c * CC, CC)
            dc = d2_ref[:, cols]
            colid = c * CC + jax.lax.broadcasted_iota(jnp.int32, (R, CC), 1)
            dc = jnp.where(colid == am_prev, BIG, dc)
            d2_ref[:, cols] = dc
            cmin = jnp.min(dc, axis=1, keepdims=True)
            carg = jnp.min(jnp.where(dc == cmin, colid, N), axis=1, keepdims=True)
            cdd = jnp.min(
                jnp.where(colid == carg, dd_ref[:, cols], BIG), axis=1, keepdims=True
            )
            better = cmin < mv
            return (
                jnp.where(better, cmin, mv),
                jnp.where(better, carg, am),
                jnp.where(better, cdd, dv),
            )

        mv0 = jnp.full((R, 1), BIG, jnp.float32)
        am0 = jnp.full((R, 1), -2, jnp.int32)
        mv, am, dv = jax.lax.fori_loop(c_lo, c_hi, scan_chunk, (mv0, am0, mv0))

        valid = mv <= CUT2
        iv = jnp.where(valid, am, rowg)
        wv = jnp.where(
            valid & (dv > 0.0), jnp.sqrt(jnp.where(dv > 0.0, dv, 1.0)), 0.0
        )
        here = slot == k
        idx_acc = jnp.where(here, iv, idx_acc)
        w_acc = jnp.where(here, wv, w_acc)
        return am, idx_acc, w_acc

    _, idx_acc, w_acc = jax.lax.fori_loop(
        0,
        K,
        select_one,
        (
            jnp.full((R, 1), -2, jnp.int32),
            jnp.zeros((R, K), jnp.int32),
            jnp.zeros((R, K), jnp.float32),
        ),
    )
    idx_out[...] = idx_acc
    w_out[...] = w_acc


def _build(N, R, CC):
    nblk = N // R
    grid_spec = pltpu.PrefetchScalarGridSpec(
        num_scalar_prefetch=2,
        grid=(nblk,),
        in_specs=[
            pl.BlockSpec((R, 8), lambda i, clo, chi: (i, 0)),
            pl.BlockSpec((8, N), lambda i, clo, chi: (0, 0)),
            pl.BlockSpec((R, 1), lambda i, clo, chi: (i, 0)),
            pl.BlockSpec((1, N), lambda i, clo, chi: (0, 0)),
        ],
        out_specs=[
            pl.BlockSpec((R, K), lambda i, clo, chi: (i, 0)),
            pl.BlockSpec((R, K), lambda i, clo, chi: (i, 0)),
        ],
        scratch_shapes=[
            pltpu.VMEM((R, N), jnp.float32),
            pltpu.VMEM((R, N), jnp.float32),
        ],
    )
    return pl.pallas_call(
        functools.partial(_tc_body, R=R, N=N, CC=CC),
        grid_spec=grid_spec,
        out_shape=[
            jax.ShapeDtypeStruct((N, K), jnp.int32),
            jax.ShapeDtypeStruct((N, K), jnp.float32),
        ],
    )


def kernel(pos, batch):
    N = pos.shape[0]
    R = 256 if N % 256 == 0 else 128
    CC = 128
    nblk = N // R
    batch32 = batch.astype(jnp.int32)
    posp = jnp.pad(pos, ((0, 0), (0, 5)))
    posTp = posp.T
    bat_col = batch32.reshape(N, 1)
    bat_row = batch32.reshape(1, N)
    r0 = jnp.arange(nblk, dtype=jnp.int32) * R
    b_first = batch32[r0]
    b_last = batch32[r0 + R - 1]
    lo = jnp.searchsorted(batch32, b_first, side="left").astype(jnp.int32)
    hi = jnp.searchsorted(batch32, b_last, side="right").astype(jnp.int32)
    clo = lo // CC
    chi = (hi + CC - 1) // CC
    idx, w = _build(N, R, CC)(clo, chi, posp, posTp, bat_col, bat_row)
    centers = jnp.broadcast_to(jnp.arange(N, dtype=jnp.int32)[:, None], (N, K))
    row = idx.reshape(-1)
    col = centers.reshape(-1)
    edge_index = jnp.stack([row, col]).astype(jnp.int64)
    edge_weight = w.reshape(-1)
    return edge_index, edge_weight


# speedup vs baseline: 13.5734x; 1.5973x over previous
import functools

import jax
import jax.numpy as jnp
from jax.experimental import pallas as pl
from jax.experimental.pallas import tpu as pltpu

CUT2 = 100.0
K = 32
BIG = 1.0e30


def _tc_body(clo_ref, chi_ref, posp_blk, posTp, bat_blk, bat_row, idx_out, w_out,
             d2_ref, dd_ref, *, R, N, CC):
    g = pl.program_id(0)
    c_lo = clo_ref[g]
    c_hi = chi_ref[g]

    xi = posp_blk[:, 0:1]
    yi = posp_blk[:, 1:2]
    zi = posp_blk[:, 2:3]
    sqi = (xi * xi + yi * yi) + zi * zi
    bi = bat_blk[:, 0:1]
    rowg = g * R + jax.lax.broadcasted_iota(jnp.int32, (R, 1), 0)

    def compute_chunk(c, _):
        cols = pl.ds(c * CC, CC)
        p = posTp[:, cols]
        xj = p[0:1, :]
        yj = p[1:2, :]
        zj = p[2:3, :]
        sqj = (xj * xj + yj * yj) + zj * zj
        gram = jnp.dot(posp_blk[...], p, preferred_element_type=jnp.float32)
        d2 = jnp.maximum((sqi + sqj) - 2.0 * gram, 0.0)
        dx = xi - xj
        dy = yi - yj
        dz = zi - zj
        dd = (dx * dx + dy * dy) + dz * dz
        colid = c * CC + jax.lax.broadcasted_iota(jnp.int32, (R, CC), 1)
        same = bi == bat_row[0:1, cols]
        keep = same & (colid != rowg)
        d2_ref[:, cols] = jnp.where(keep, d2, BIG)
        dd_ref[:, cols] = dd
        return 0

    jax.lax.fori_loop(c_lo, c_hi, compute_chunk, 0)

    slot = jax.lax.broadcasted_iota(jnp.int32, (R, K), 1)

    def select_one(k, state):
        am_prev, idx_acc, w_acc = state

        def scan_chunk(c, carry):
            mv, am, dv = carry
            cols = pl.ds(c * CC, CC)
            dc = d2_ref[:, cols]
            colid = c * CC + jax.lax.broadcasted_iota(jnp.int32, (R, CC), 1)
            dc = jnp.where(colid == am_prev, BIG, dc)
            d2_ref[:, cols] = dc
            cmin = jnp.min(dc, axis=1, keepdims=True)
            carg = jnp.min(jnp.where(dc == cmin, colid, N), axis=1, keepdims=True)
            cdd = jnp.min(
                jnp.where(colid == carg, dd_ref[:, cols], BIG), axis=1, keepdims=True
            )
            better = cmin < mv
            return (
                jnp.where(better, cmin, mv),
                jnp.where(better, carg, am),
                jnp.where(better, cdd, dv),
            )

        mv0 = jnp.full((R, 1), BIG, jnp.float32)
        am0 = jnp.full((R, 1), -2, jnp.int32)
        mv, am, dv = jax.lax.fori_loop(c_lo, c_hi, scan_chunk, (mv0, am0, mv0))

        valid = mv <= CUT2
        iv = jnp.where(valid, am, rowg)
        wv = jnp.where(
            valid & (dv > 0.0), jnp.sqrt(jnp.where(dv > 0.0, dv, 1.0)), 0.0
        )
        here = slot == k
        idx_acc = jnp.where(here, iv, idx_acc)
        w_acc = jnp.where(here, wv, w_acc)
        return am, idx_acc, w_acc

    _, idx_acc, w_acc = jax.lax.fori_loop(
        0,
        K,
        select_one,
        (
            jnp.full((R, 1), -2, jnp.int32),
            jnp.zeros((R, K), jnp.int32),
            jnp.zeros((R, K), jnp.float32),
        ),
    )
    idx_out[...] = idx_acc
    w_out[...] = w_acc


def _build(N, R, CC):
    nblk = N // R
    grid_spec = pltpu.PrefetchScalarGridSpec(
        num_scalar_prefetch=2,
        grid=(nblk,),
        in_specs=[
            pl.BlockSpec((R, 8), lambda i, clo, chi: (i, 0)),
            pl.BlockSpec((8, N), lambda i, clo, chi: (0, 0)),
            pl.BlockSpec((R, 1), lambda i, clo, chi: (i, 0)),
            pl.BlockSpec((1, N), lambda i, clo, chi: (0, 0)),
        ],
        out_specs=[
            pl.BlockSpec((R, K), lambda i, clo, chi: (i, 0)),
            pl.BlockSpec((R, K), lambda i, clo, chi: (i, 0)),
        ],
        scratch_shapes=[
            pltpu.VMEM((R, N), jnp.float32),
            pltpu.VMEM((R, N), jnp.float32),
        ],
    )
    return pl.pallas_call(
        functools.partial(_tc_body, R=R, N=N, CC=CC),
        grid_spec=grid_spec,
        out_shape=[
            jax.ShapeDtypeStruct((N, K), jnp.int32),
            jax.ShapeDtypeStruct((N, K), jnp.float32),
        ],
    )


def kernel(pos, batch):
    N = pos.shape[0]
    R = 256 if N % 256 == 0 else 128
    CC = 256
    nblk = N // R
    batch32 = batch.astype(jnp.int32)
    posp = jnp.pad(pos, ((0, 0), (0, 5)))
    posTp = posp.T
    bat_col = batch32.reshape(N, 1)
    bat_row = batch32.reshape(1, N)
    r0 = jnp.arange(nblk, dtype=jnp.int32) * R
    b_first = batch32[r0]
    b_last = batch32[r0 + R - 1]
    lo = jnp.searchsorted(batch32, b_first, side="left").astype(jnp.int32)
    hi = jnp.searchsorted(batch32, b_last, side="right").astype(jnp.int32)
    clo = lo // CC
    chi = (hi + CC - 1) // CC
    idx, w = _build(N, R, CC)(clo, chi, posp, posTp, bat_col, bat_row)
    centers = jnp.broadcast_to(jnp.arange(N, dtype=jnp.int32)[:, None], (N, K))
    row = idx.reshape(-1)
    col = centers.reshape(-1)
    edge_index = jnp.stack([row, col]).astype(jnp.int64)
    edge_weight = w.reshape(-1)
    return edge_index, edge_weight


# speedup vs baseline: 18.5975x; 1.3701x over previous
import functools

import jax
import jax.numpy as jnp
from jax.experimental import pallas as pl
from jax.experimental.pallas import tpu as pltpu

CUT2 = 100.0
K = 32
BIG = 1.0e30


def _tc_body(clo_ref, chi_ref, posp_blk, posTp, bat_blk, bat_row, idx_out, w_out,
             d2_ref, dd_ref, *, R, N, CC):
    g = pl.program_id(0)
    c_lo = clo_ref[g]
    c_hi = chi_ref[g]

    xi = posp_blk[:, 0:1]
    yi = posp_blk[:, 1:2]
    zi = posp_blk[:, 2:3]
    sqi = (xi * xi + yi * yi) + zi * zi
    bi = bat_blk[:, 0:1]
    rowg = g * R + jax.lax.broadcasted_iota(jnp.int32, (R, 1), 0)

    def compute_chunk(c, _):
        cols = pl.ds(c * CC, CC)
        p = posTp[:, cols]
        xj = p[0:1, :]
        yj = p[1:2, :]
        zj = p[2:3, :]
        sqj = (xj * xj + yj * yj) + zj * zj
        gram = jnp.dot(posp_blk[...], p, preferred_element_type=jnp.float32)
        d2 = jnp.maximum((sqi + sqj) - 2.0 * gram, 0.0)
        dx = xi - xj
        dy = yi - yj
        dz = zi - zj
        dd = (dx * dx + dy * dy) + dz * dz
        colid = c * CC + jax.lax.broadcasted_iota(jnp.int32, (R, CC), 1)
        same = bi == bat_row[0:1, cols]
        keep = same & (colid != rowg)
        d2_ref[:, cols] = jnp.where(keep, d2, BIG)
        dd_ref[:, cols] = dd
        return 0

    jax.lax.fori_loop(c_lo, c_hi, compute_chunk, 0)

    slot = jax.lax.broadcasted_iota(jnp.int32, (R, K), 1)

    def select_one(k, state):
        am_prev, idx_acc, w_acc = state

        def scan_chunk(c, carry):
            mv, am, dv = carry
            cols = pl.ds(c * CC, CC)
            dc = d2_ref[:, cols]
            colid = c * CC + jax.lax.broadcasted_iota(jnp.int32, (R, CC), 1)
            dc = jnp.where(colid == am_prev, BIG, dc)
            d2_ref[:, cols] = dc
            cmin = jnp.min(dc, axis=1, keepdims=True)
            carg = jnp.min(jnp.where(dc == cmin, colid, N), axis=1, keepdims=True)
            cdd = jnp.min(
                jnp.where(colid == carg, dd_ref[:, cols], BIG), axis=1, keepdims=True
            )
            better = cmin < mv
            return (
                jnp.where(better, cmin, mv),
                jnp.where(better, carg, am),
                jnp.where(better, cdd, dv),
            )

        mv0 = jnp.full((R, 1), BIG, jnp.float32)
        am0 = jnp.full((R, 1), -2, jnp.int32)
        mv, am, dv = jax.lax.fori_loop(c_lo, c_hi, scan_chunk, (mv0, am0, mv0))

        valid = mv <= CUT2
        iv = jnp.where(valid, am, rowg)
        wv = jnp.where(
            valid & (dv > 0.0), jnp.sqrt(jnp.where(dv > 0.0, dv, 1.0)), 0.0
        )
        here = slot == k
        idx_acc = jnp.where(here, iv, idx_acc)
        w_acc = jnp.where(here, wv, w_acc)
        return am, idx_acc, w_acc

    _, idx_acc, w_acc = jax.lax.fori_loop(
        0,
        K,
        select_one,
        (
            jnp.full((R, 1), -2, jnp.int32),
            jnp.zeros((R, K), jnp.int32),
            jnp.zeros((R, K), jnp.float32),
        ),
    )
    idx_out[...] = idx_acc
    w_out[...] = w_acc


def _build(N, R, CC):
    nblk = N // R
    grid_spec = pltpu.PrefetchScalarGridSpec(
        num_scalar_prefetch=2,
        grid=(nblk,),
        in_specs=[
            pl.BlockSpec((R, 8), lambda i, clo, chi: (i, 0)),
            pl.BlockSpec((8, N), lambda i, clo, chi: (0, 0)),
            pl.BlockSpec((R, 1), lambda i, clo, chi: (i, 0)),
            pl.BlockSpec((1, N), lambda i, clo, chi: (0, 0)),
        ],
        out_specs=[
            pl.BlockSpec((R, K), lambda i, clo, chi: (i, 0)),
            pl.BlockSpec((R, K), lambda i, clo, chi: (i, 0)),
        ],
        scratch_shapes=[
            pltpu.VMEM((R, N), jnp.float32),
            pltpu.VMEM((R, N), jnp.float32),
        ],
    )
    return pl.pallas_call(
        functools.partial(_tc_body, R=R, N=N, CC=CC),
        grid_spec=grid_spec,
        out_shape=[
            jax.ShapeDtypeStruct((N, K), jnp.int32),
            jax.ShapeDtypeStruct((N, K), jnp.float32),
        ],
    )


def kernel(pos, batch):
    N = pos.shape[0]
    R = 256 if N % 256 == 0 else 128
    CC = 512
    nblk = N // R
    batch32 = batch.astype(jnp.int32)
    posp = jnp.pad(pos, ((0, 0), (0, 5)))
    posTp = posp.T
    bat_col = batch32.reshape(N, 1)
    bat_row = batch32.reshape(1, N)
    r0 = jnp.arange(nblk, dtype=jnp.int32) * R
    b_first = batch32[r0]
    b_last = batch32[r0 + R - 1]
    lo = jnp.searchsorted(batch32, b_first, side="left").astype(jnp.int32)
    hi = jnp.searchsorted(batch32, b_last, side="right").astype(jnp.int32)
    clo = lo // CC
    chi = (hi + CC - 1) // CC
    idx, w = _build(N, R, CC)(clo, chi, posp, posTp, bat_col, bat_row)
    centers = jnp.broadcast_to(jnp.arange(N, dtype=jnp.int32)[:, None], (N, K))
    row = idx.reshape(-1)
    col = centers.reshape(-1)
    edge_index = jnp.stack([row, col]).astype(jnp.int64)
    edge_weight = w.reshape(-1)
    return edge_index, edge_weight


# speedup vs baseline: 19.8680x; 1.0683x over previous
import functools

import jax
import jax.numpy as jnp
from jax.experimental import pallas as pl
from jax.experimental.pallas import tpu as pltpu

CUT2 = 100.0
K = 32
BIG = 1.0e30


def _tc_body(clo_ref, chi_ref, posp_blk, posTp, bat_blk, bat_row, idx_out, w_out,
             d2_ref, dd_ref, *, R, N, CC):
    g = pl.program_id(0)
    c_lo = clo_ref[g]
    c_hi = chi_ref[g]

    xi = posp_blk[:, 0:1]
    yi = posp_blk[:, 1:2]
    zi = posp_blk[:, 2:3]
    sqi = (xi * xi + yi * yi) + zi * zi
    bi = bat_blk[:, 0:1]
    rowg = g * R + jax.lax.broadcasted_iota(jnp.int32, (R, 1), 0)

    def compute_chunk(c, _):
        cols = pl.ds(c * CC, CC)
        p = posTp[:, cols]
        xj = p[0:1, :]
        yj = p[1:2, :]
        zj = p[2:3, :]
        sqj = (xj * xj + yj * yj) + zj * zj
        gram = jnp.dot(posp_blk[...], p, preferred_element_type=jnp.float32)
        d2 = jnp.maximum((sqi + sqj) - 2.0 * gram, 0.0)
        dx = xi - xj
        dy = yi - yj
        dz = zi - zj
        dd = (dx * dx + dy * dy) + dz * dz
        colid = c * CC + jax.lax.broadcasted_iota(jnp.int32, (R, CC), 1)
        same = bi == bat_row[0:1, cols]
        keep = same & (colid != rowg)
        d2_ref[:, cols] = jnp.where(keep, d2, BIG)
        dd_ref[:, cols] = dd
        return 0

    jax.lax.fori_loop(c_lo, c_hi, compute_chunk, 0)

    slot = jax.lax.broadcasted_iota(jnp.int32, (R, K), 1)

    def select_one(k, state):
        am_prev, idx_acc, w_acc = state

        def scan_chunk(c, carry):
            mv, am, dv = carry
            cols = pl.ds(c * CC, CC)
            dc = d2_ref[:, cols]
            colid = c * CC + jax.lax.broadcasted_iota(jnp.int32, (R, CC), 1)
            dc = jnp.where(colid == am_prev, BIG, dc)
            d2_ref[:, cols] = dc
            cmin = jnp.min(dc, axis=1, keepdims=True)
            carg = jnp.min(jnp.where(dc == cmin, colid, N), axis=1, keepdims=True)
            cdd = jnp.min(
                jnp.where(colid == carg, dd_ref[:, cols], BIG), axis=1, keepdims=True
            )
            better = cmin < mv
            return (
                jnp.where(better, cmin, mv),
                jnp.where(better, carg, am),
                jnp.where(better, cdd, dv),
            )

        mv0 = jnp.full((R, 1), BIG, jnp.float32)
        am0 = jnp.full((R, 1), -2, jnp.int32)
        mv, am, dv = jax.lax.fori_loop(c_lo, c_hi, scan_chunk, (mv0, am0, mv0))

        valid = mv <= CUT2
        iv = jnp.where(valid, am, rowg)
        wv = jnp.where(
            valid & (dv > 0.0), jnp.sqrt(jnp.where(dv > 0.0, dv, 1.0)), 0.0
        )
        here = slot == k
        idx_acc = jnp.where(here, iv, idx_acc)
        w_acc = jnp.where(here, wv, w_acc)
        return am, idx_acc, w_acc

    _, idx_acc, w_acc = jax.lax.fori_loop(
        0,
        K,
        select_one,
        (
            jnp.full((R, 1), -2, jnp.int32),
            jnp.zeros((R, K), jnp.int32),
            jnp.zeros((R, K), jnp.float32),
        ),
    )
    idx_out[...] = idx_acc
    w_out[...] = w_acc


def _build(N, R, CC):
    nblk = N // R
    grid_spec = pltpu.PrefetchScalarGridSpec(
        num_scalar_prefetch=2,
        grid=(nblk,),
        in_specs=[
            pl.BlockSpec((R, 8), lambda i, clo, chi: (i, 0)),
            pl.BlockSpec((8, N), lambda i, clo, chi: (0, 0)),
            pl.BlockSpec((R, 1), lambda i, clo, chi: (i, 0)),
            pl.BlockSpec((1, N), lambda i, clo, chi: (0, 0)),
        ],
        out_specs=[
            pl.BlockSpec((R, K), lambda i, clo, chi: (i, 0)),
            pl.BlockSpec((R, K), lambda i, clo, chi: (i, 0)),
        ],
        scratch_shapes=[
            pltpu.VMEM((R, N), jnp.float32),
            pltpu.VMEM((R, N), jnp.float32),
        ],
    )
    return pl.pallas_call(
        functools.partial(_tc_body, R=R, N=N, CC=CC),
        grid_spec=grid_spec,
        out_shape=[
            jax.ShapeDtypeStruct((N, K), jnp.int32),
            jax.ShapeDtypeStruct((N, K), jnp.float32),
        ],
    )


def kernel(pos, batch):
    N = pos.shape[0]
    R = 256 if N % 256 == 0 else 128
    CC = 1024
    nblk = N // R
    batch32 = batch.astype(jnp.int32)
    posp = jnp.pad(pos, ((0, 0), (0, 5)))
    posTp = posp.T
    bat_col = batch32.reshape(N, 1)
    bat_row = batch32.reshape(1, N)
    r0 = jnp.arange(nblk, dtype=jnp.int32) * R
    b_first = batch32[r0]
    b_last = batch32[r0 + R - 1]
    lo = jnp.searchsorted(batch32, b_first, side="left").astype(jnp.int32)
    hi = jnp.searchsorted(batch32, b_last, side="right").astype(jnp.int32)
    clo = lo // CC
    chi = (hi + CC - 1) // CC
    idx, w = _build(N, R, CC)(clo, chi, posp, posTp, bat_col, bat_row)
    centers = jnp.broadcast_to(jnp.arange(N, dtype=jnp.int32)[:, None], (N, K))
    row = idx.reshape(-1)
    col = centers.reshape(-1)
    edge_index = jnp.stack([row, col]).astype(jnp.int64)
    edge_weight = w.reshape(-1)
    return edge_index, edge_weight
